# baseline (speedup 1.0000x reference)
import functools

import jax
import jax.numpy as jnp
from jax import lax
from jax.experimental import pallas as pl
from jax.experimental.pallas import tpu as pltpu
from jax.experimental.pallas import tpu_sc as plsc

_NSEM = 4


@functools.lru_cache(maxsize=None)
def _make_gather(B, V, D):
    info = plsc.get_sparse_core_info()
    NC, NS = info.num_cores, info.num_subcores
    NW = NC * NS
    assert B % (8 * NW) == 0
    b_per_w = B // NW

    mesh = plsc.VectorSubcoreMesh(core_axis_name="c", subcore_axis_name="s")

    @functools.partial(
        pl.kernel,
        mesh=mesh,
        out_type=jax.ShapeDtypeStruct((B, D), jnp.float32),
        scratch_types=[
            pltpu.VMEM((b_per_w,), jnp.int32),
            pltpu.VMEM((b_per_w, D), jnp.float32),
            [pltpu.SemaphoreType.DMA] * _NSEM,
        ],
    )
    def gather_kernel(table_hbm, idx_hbm, out_hbm, idx_v, rows_v, sems):
        wid = lax.axis_index("s") * NC + lax.axis_index("c")
        base = wid * b_per_w
        pltpu.sync_copy(idx_hbm.at[pl.ds(base, b_per_w)], idx_v)
        L = 16

        def body(g, carry):
            vec = idx_v[pl.ds(g * L, L)]
            for j in range(L):
                k = g * L + j
                pltpu.async_copy(
                    table_hbm.at[pl.ds(vec[j], 1)],
                    rows_v.at[pl.ds(k, 1)],
                    sems[j % _NSEM],
                )
            return carry

        lax.fori_loop(0, b_per_w // L, body, 0)
        per_sem = b_per_w // _NSEM
        for q in range(_NSEM):
            pltpu.make_async_copy(
                table_hbm.at[pl.ds(0, per_sem)],
                rows_v.at[pl.ds(q * per_sem, per_sem)],
                sems[q],
            ).wait()
        pltpu.sync_copy(rows_v, out_hbm.at[pl.ds(base, b_per_w)])

    return gather_kernel


def kernel(indices, mean, logstd):
    B, = indices.shape
    V, D = mean.shape
    idx = jnp.asarray(indices, jnp.int32)
    return _make_gather(B, V, D)(mean, idx)
